# baseline (speedup 1.0000x reference)
import functools

import jax
import jax.numpy as jnp
from jax.experimental import pallas as pl
from jax.experimental.pallas import tpu as pltpu

_ALPHA = 0.25
_DELTA = 0.1
_AB = 8192


def _focal(ce):
    p = jnp.exp(-ce)
    return _ALPHA * (1.0 - p) * (1.0 - p) * ce


def _loss_body(cls_r, tb_r, gb_r, to_r, gc_r, go_r, cls_o, obj_o, bb_o):
    j = pl.program_id(0)

    @pl.when(j == 0)
    def _():
        cls_o[0, 0] = 0.0
        obj_o[0, 0] = 0.0
        bb_o[0, 0] = 0.0

    cls_acc = jnp.zeros((1, _AB), dtype=jnp.float32)
    obj_acc = jnp.zeros((1, _AB), dtype=jnp.float32)
    bb_acc = jnp.zeros((1, _AB), dtype=jnp.float32)
    for bi in range(cls_r.shape[0]):
        go = go_r[bi:bi + 1, :]
        gc = gc_r[bi:bi + 1, :]
        mask_obj = (go != -1).astype(jnp.float32)
        mask_bb = (go == 1).astype(jnp.float32)

        x = cls_r[bi]
        s = jnp.sum(jnp.exp(x), axis=0, keepdims=True)
        oh = jax.lax.broadcasted_iota(jnp.int32, x.shape, 0) == gc
        sel = jnp.sum(jnp.where(oh, x, 0.0), axis=0, keepdims=True)
        ce = jnp.log(s) - sel
        cls_acc += _focal(ce) * mask_bb

        t = to_r[bi]
        a = t[0:1, :]
        b = t[1:2, :]
        s2 = jnp.exp(a) + jnp.exp(b)
        sel2 = jnp.where(go == 1, b, a)
        ce2 = jnp.log(s2) - sel2
        obj_acc += _focal(ce2) * mask_obj

        d = tb_r[bi] - gb_r[bi]
        ad = jnp.abs(d)
        sl1 = jnp.where(ad < _DELTA, (0.5 / _DELTA) * d * d,
                        ad - 0.5 * _DELTA)
        bb_acc += jnp.sum(sl1, axis=0, keepdims=True) * mask_bb

    cls_o[0, 0] += jnp.sum(cls_acc)
    obj_o[0, 0] += jnp.sum(obj_acc)
    bb_o[0, 0] += jnp.sum(bb_acc)


@functools.partial(jax.jit, static_argnames=("interpret",))
def _loss_sums(clsT, tbT, gbT, toT, gc2, go2, interpret=False):
    bsz, c, a = clsT.shape
    nj = a // _AB
    scalar_spec = pl.BlockSpec((1, 1), lambda j: (0, 0),
                               memory_space=pltpu.SMEM)
    return pl.pallas_call(
        _loss_body,
        grid=(nj,),
        in_specs=[
            pl.BlockSpec((bsz, c, _AB), lambda j: (0, 0, j)),
            pl.BlockSpec((bsz, 4, _AB), lambda j: (0, 0, j)),
            pl.BlockSpec((bsz, 4, _AB), lambda j: (0, 0, j)),
            pl.BlockSpec((bsz, 2, _AB), lambda j: (0, 0, j)),
            pl.BlockSpec((bsz, _AB), lambda j: (0, j)),
            pl.BlockSpec((bsz, _AB), lambda j: (0, j)),
        ],
        out_specs=[scalar_spec, scalar_spec, scalar_spec],
        out_shape=[jax.ShapeDtypeStruct((1, 1), jnp.float32)] * 3,
        compiler_params=pltpu.CompilerParams(
            dimension_semantics=("arbitrary",)),
        interpret=interpret,
    )(clsT, tbT, gbT, toT, gc2, go2)


def kernel(targets_bb, targets_cls, targets_obj, gt_targets_bb,
           gt_targets_cls, gt_targets_obj, w_obj, w_cls, w_bb, step,
           interpret=False):
    n = targets_cls.shape[0] * targets_cls.shape[1]
    clsT = jnp.transpose(targets_cls, (0, 2, 1))
    tbT = jnp.transpose(targets_bb, (0, 2, 1))
    gbT = jnp.transpose(gt_targets_bb, (0, 2, 1))
    toT = jnp.transpose(targets_obj, (0, 2, 1))
    cls_s, obj_s, bb_s = _loss_sums(clsT, tbT, gbT, toT,
                                    gt_targets_cls, gt_targets_obj,
                                    interpret=interpret)
    inv_n = 1.0 / jnp.float32(n)
    cls_loss = cls_s[0, 0] * inv_n * 10000.0
    obj_loss = obj_s[0, 0] * inv_n * 5000.0
    bb_loss = bb_s[0, 0] * inv_n * 20000.0
    cls_loss = cls_loss * jnp.exp(-w_cls) + w_cls
    obj_loss = obj_loss * jnp.exp(-w_obj) + w_obj
    bb_loss = bb_loss * jnp.exp(-w_bb) + w_bb
    return (cls_loss, obj_loss, bb_loss)
